# Initial kernel scaffold; baseline (speedup 1.0000x reference)
import jax
import jax.numpy as jnp
from jax.experimental import pallas as pl


def kernel(conv1_w, conv1_b, conv2_w, conv2_b, fc1_w, fc1_b, fc2_w, fc2_b, fc3_w, fc3_b, img):
    raise NotImplementedError("write your pallas kernel here")


# speedup vs baseline: 28.6498x; 28.6498x over previous
import jax
import jax.numpy as jnp
from jax.experimental import pallas as pl
from jax.experimental.pallas import tpu as pltpu


_TILE_B = 256
_VMEM_LIMIT = 48 * 1024 * 1024


def _pool_corner_toeplitz(in_size, out_size, d):
    ih = jnp.arange(in_size)[:, None, None]
    p = jnp.arange(out_size)[None, :, None]
    k = jnp.arange(5)[None, None, :]
    return (ih == 2 * p + d + k).astype(jnp.float32)


def _conv1_dense(conv1_w):
    w = conv1_w.reshape(5, 5, 6)
    blocks = []
    for di in (0, 1):
        th = _pool_corner_toeplitz(28, 12, di)
        for dj in (0, 1):
            tw = _pool_corner_toeplitz(28, 12, dj)
            blk = jnp.einsum("hpi,wqj,ijc->hwpqc", th, tw, w).reshape(784, 864)
            blocks.append(jnp.pad(blk, ((0, 0), (0, 32))))
    return jnp.concatenate(blocks, axis=1)


def _conv2_dense(conv2_w):
    w = conv2_w.reshape(6, 5, 5, 16)
    blocks = []
    for di in (0, 1):
        th = _pool_corner_toeplitz(12, 4, di)
        for dj in (0, 1):
            tw = _pool_corner_toeplitz(12, 4, dj)
            blk = jnp.einsum("hpi,wqj,cijo->hwcpqo", th, tw, w).reshape(864, 256)
            blocks.append(blk)
    m = jnp.concatenate(blocks, axis=1)
    return jnp.pad(m, ((0, 32), (0, 0)))


def _lenet_kernel(x_ref, w1_ref, b1_ref, w2_ref, b2_ref,
                  f1w_ref, f1b_ref, f2w_ref, f2b_ref, f3w_ref, f3b_ref,
                  o_ref):
    xb = x_ref[...].astype(jnp.bfloat16)
    s = jnp.dot(xb, w1_ref[...], preferred_element_type=jnp.float32)
    m = jnp.maximum(jnp.maximum(s[:, 0:896], s[:, 896:1792]),
                    jnp.maximum(s[:, 1792:2688], s[:, 2688:3584]))
    h = jax.nn.sigmoid(m + b1_ref[...]).astype(jnp.bfloat16)

    s2 = jnp.dot(h, w2_ref[...], preferred_element_type=jnp.float32)
    m2 = jnp.maximum(jnp.maximum(s2[:, 0:256], s2[:, 256:512]),
                     jnp.maximum(s2[:, 512:768], s2[:, 768:1024]))
    h2 = jax.nn.sigmoid(m2 + b2_ref[...]).astype(jnp.bfloat16)

    h3 = jax.nn.sigmoid(
        jnp.dot(h2, f1w_ref[...], preferred_element_type=jnp.float32)
        + f1b_ref[...]).astype(jnp.bfloat16)
    h4 = jax.nn.sigmoid(
        jnp.dot(h3, f2w_ref[...], preferred_element_type=jnp.float32)
        + f2b_ref[...]).astype(jnp.bfloat16)
    out = (jnp.dot(h4, f3w_ref[...], preferred_element_type=jnp.float32)
           + f3b_ref[...])
    o_ref[...] = out.astype(o_ref.dtype)


def kernel(conv1_w, conv1_b, conv2_w, conv2_b, fc1_w, fc1_b,
           fc2_w, fc2_b, fc3_w, fc3_b, img):
    B = img.shape[0]
    x = img.reshape(B, 28 * 28)

    w1 = _conv1_dense(conv1_w).astype(jnp.bfloat16)
    w2 = _conv2_dense(conv2_w).astype(jnp.bfloat16)
    b1 = jnp.pad(jnp.tile(conv1_b, (1, 144)), ((0, 0), (0, 32)))
    b2 = jnp.tile(conv2_b, (1, 16))
    f1w = fc1_w.astype(jnp.bfloat16)
    f2w = fc2_w.astype(jnp.bfloat16)
    f3w = fc3_w.astype(jnp.bfloat16)

    tile_b = B if B <= _TILE_B else _TILE_B
    grid = (pl.cdiv(B, tile_b),)
    cost = pl.CostEstimate(
        flops=2 * B * (784 * 3584 + 896 * 1024 + 256 * 120 + 120 * 84 + 84 * 10),
        transcendentals=B * (896 + 256 + 120 + 84),
        bytes_accessed=4 * B * (784 + 10) + 2 * (784 * 3584 + 896 * 1024),
    )
    const = lambda i: (0, 0)
    out = pl.pallas_call(
        _lenet_kernel,
        out_shape=jax.ShapeDtypeStruct((B, 10), jnp.float32),
        grid=grid,
        in_specs=[
            pl.BlockSpec((tile_b, 784), lambda i: (i, 0)),
            pl.BlockSpec((784, 3584), const),
            pl.BlockSpec((1, 896), const),
            pl.BlockSpec((896, 1024), const),
            pl.BlockSpec((1, 256), const),
            pl.BlockSpec((256, 120), const),
            pl.BlockSpec((1, 120), const),
            pl.BlockSpec((120, 84), const),
            pl.BlockSpec((1, 84), const),
            pl.BlockSpec((84, 10), const),
            pl.BlockSpec((1, 10), const),
        ],
        out_specs=pl.BlockSpec((tile_b, 10), lambda i: (i, 0)),
        compiler_params=pltpu.CompilerParams(
            dimension_semantics=("parallel",),
            vmem_limit_bytes=_VMEM_LIMIT,
        ),
        cost_estimate=cost,
    )(x, w1, b1, w2, b2, f1w, fc1_b, f2w, fc2_b, f3w, fc3_b)
    return out


# speedup vs baseline: 30.3857x; 1.0606x over previous
import jax
import jax.numpy as jnp
from jax.experimental import pallas as pl
from jax.experimental.pallas import tpu as pltpu


_TILE_B = 512
_VMEM_LIMIT = 48 * 1024 * 1024


def _pool_corner_toeplitz(in_size, out_size, d):
    ih = jnp.arange(in_size)[:, None, None]
    p = jnp.arange(out_size)[None, :, None]
    k = jnp.arange(5)[None, None, :]
    return (ih == 2 * p + d + k).astype(jnp.float32)


def _conv1_dense(conv1_w):
    w = conv1_w.reshape(5, 5, 6)
    blocks = []
    for di in (0, 1):
        th = _pool_corner_toeplitz(28, 12, di)
        for dj in (0, 1):
            tw = _pool_corner_toeplitz(28, 12, dj)
            blk = jnp.einsum("hpi,wqj,ijc->hwpqc", th, tw, w).reshape(784, 864)
            blocks.append(jnp.pad(blk, ((0, 0), (0, 32))))
    return jnp.concatenate(blocks, axis=1)


def _conv2_dense(conv2_w):
    w = conv2_w.reshape(6, 5, 5, 16)
    blocks = []
    for di in (0, 1):
        th = _pool_corner_toeplitz(12, 4, di)
        for dj in (0, 1):
            tw = _pool_corner_toeplitz(12, 4, dj)
            blk = jnp.einsum("hpi,wqj,cijo->hwcpqo", th, tw, w).reshape(864, 256)
            blocks.append(blk)
    m = jnp.concatenate(blocks, axis=1)
    return jnp.pad(m, ((0, 32), (0, 0)))


def _lenet_kernel(x_ref, w1_ref, b1_ref, w2_ref, b2_ref,
                  f1w_ref, f1b_ref, f2w_ref, f2b_ref, f3w_ref, f3b_ref,
                  o_ref):
    xb = x_ref[...].astype(jnp.bfloat16)
    s = jnp.dot(xb, w1_ref[...], preferred_element_type=jnp.float32)
    m = jnp.maximum(jnp.maximum(s[:, 0:896], s[:, 896:1792]),
                    jnp.maximum(s[:, 1792:2688], s[:, 2688:3584]))
    h = jax.nn.sigmoid(m + b1_ref[...]).astype(jnp.bfloat16)

    s2 = jnp.dot(h, w2_ref[...], preferred_element_type=jnp.float32)
    m2 = jnp.maximum(jnp.maximum(s2[:, 0:256], s2[:, 256:512]),
                     jnp.maximum(s2[:, 512:768], s2[:, 768:1024]))
    h2 = jax.nn.sigmoid(m2 + b2_ref[...]).astype(jnp.bfloat16)

    h3 = jax.nn.sigmoid(
        jnp.dot(h2, f1w_ref[...], preferred_element_type=jnp.float32)
        + f1b_ref[...]).astype(jnp.bfloat16)
    h4 = jax.nn.sigmoid(
        jnp.dot(h3, f2w_ref[...], preferred_element_type=jnp.float32)
        + f2b_ref[...]).astype(jnp.bfloat16)
    out = (jnp.dot(h4, f3w_ref[...], preferred_element_type=jnp.float32)
           + f3b_ref[...])
    o_ref[...] = out.astype(o_ref.dtype)


def kernel(conv1_w, conv1_b, conv2_w, conv2_b, fc1_w, fc1_b,
           fc2_w, fc2_b, fc3_w, fc3_b, img):
    B = img.shape[0]
    x = img.reshape(B, 28 * 28)

    w1 = _conv1_dense(conv1_w).astype(jnp.bfloat16)
    w2 = _conv2_dense(conv2_w).astype(jnp.bfloat16)
    b1 = jnp.pad(jnp.tile(conv1_b, (1, 144)), ((0, 0), (0, 32)))
    b2 = jnp.tile(conv2_b, (1, 16))
    f1w = fc1_w.astype(jnp.bfloat16)
    f2w = fc2_w.astype(jnp.bfloat16)
    f3w = fc3_w.astype(jnp.bfloat16)

    tile_b = B if B <= _TILE_B else _TILE_B
    grid = (pl.cdiv(B, tile_b),)
    cost = pl.CostEstimate(
        flops=2 * B * (784 * 3584 + 896 * 1024 + 256 * 120 + 120 * 84 + 84 * 10),
        transcendentals=B * (896 + 256 + 120 + 84),
        bytes_accessed=4 * B * (784 + 10) + 2 * (784 * 3584 + 896 * 1024),
    )
    const = lambda i: (0, 0)
    out = pl.pallas_call(
        _lenet_kernel,
        out_shape=jax.ShapeDtypeStruct((B, 10), jnp.float32),
        grid=grid,
        in_specs=[
            pl.BlockSpec((tile_b, 784), lambda i: (i, 0)),
            pl.BlockSpec((784, 3584), const),
            pl.BlockSpec((1, 896), const),
            pl.BlockSpec((896, 1024), const),
            pl.BlockSpec((1, 256), const),
            pl.BlockSpec((256, 120), const),
            pl.BlockSpec((1, 120), const),
            pl.BlockSpec((120, 84), const),
            pl.BlockSpec((1, 84), const),
            pl.BlockSpec((84, 10), const),
            pl.BlockSpec((1, 10), const),
        ],
        out_specs=pl.BlockSpec((tile_b, 10), lambda i: (i, 0)),
        compiler_params=pltpu.CompilerParams(
            dimension_semantics=("parallel",),
            vmem_limit_bytes=_VMEM_LIMIT,
        ),
        cost_estimate=cost,
    )(x, w1, b1, w2, b2, f1w, fc1_b, f2w, fc2_b, f3w, fc3_b)
    return out
